# Initial kernel scaffold; baseline (speedup 1.0000x reference)
import jax
import jax.numpy as jnp
from jax.experimental import pallas as pl


def kernel(x_features, atoms, edge_attr, pos, atom_table, edge_table, W, b):
    raise NotImplementedError("write your pallas kernel here")


# speedup vs baseline: 2.2027x; 2.2027x over previous
import functools

import jax
import jax.numpy as jnp
from jax import lax
from jax.experimental import pallas as pl
from jax.experimental.pallas import tpu as pltpu
from jax.experimental.pallas import tpu_sc as plsc

NODE_DIM = 128
FT_DIM = 128
EDGE_DIM = 16
N_NODES = 100000
N_EDGES = 1600000
N_ATOM = 100

BN = 2000
NB = N_NODES // BN

CHUNK = 128
N_CHUNKS = N_EDGES // CHUNK
NW = 32
BASE_CHUNKS = N_CHUNKS // NW
EXTRA = N_CHUNKS - BASE_CHUNKS * NW


def _node_body(x_ref, atoms_ref, at_ref, w_ref, b_ref, o_ref):
    a = atoms_ref[0, 0, :]
    onehot = (a[:, None] == lax.broadcasted_iota(jnp.int32, (BN, N_ATOM), 1)
              ).astype(jnp.float32)
    w = w_ref[...]
    p = lax.dot_general(at_ref[...], w[:, :NODE_DIM],
                        (((1,), (1,)), ((), ())),
                        preferred_element_type=jnp.float32)
    t1 = lax.dot_general(onehot, p, (((1,), (0,)), ((), ())),
                         preferred_element_type=jnp.float32)
    t2 = lax.dot_general(x_ref[...], w[:, NODE_DIM:],
                         (((1,), (1,)), ((), ())),
                         preferred_element_type=jnp.float32)
    o_ref[...] = t1 + t2 + b_ref[...]


def _node_proj(x_features, atoms, atom_table, W, b):
    atoms3 = atoms.reshape(NB, 1, BN)
    b2 = b.reshape(1, NODE_DIM)
    return pl.pallas_call(
        _node_body,
        grid=(NB,),
        in_specs=[
            pl.BlockSpec((BN, FT_DIM), lambda i: (i, 0)),
            pl.BlockSpec((1, 1, BN), lambda i: (i, 0, 0)),
            pl.BlockSpec((N_ATOM, NODE_DIM), lambda i: (0, 0)),
            pl.BlockSpec((NODE_DIM, NODE_DIM + FT_DIM), lambda i: (0, 0)),
            pl.BlockSpec((1, NODE_DIM), lambda i: (0, 0)),
        ],
        out_specs=pl.BlockSpec((BN, NODE_DIM), lambda i: (i, 0)),
        out_shape=jax.ShapeDtypeStruct((N_NODES, NODE_DIM), jnp.float32),
    )(x_features, atoms3, atom_table, W, b2)


def _edge_gather(edge_table, edge_attr):
    idx2d = edge_attr.reshape(N_CHUNKS, CHUNK)
    mesh = plsc.VectorSubcoreMesh(core_axis_name="c", subcore_axis_name="s")

    @functools.partial(
        pl.kernel,
        mesh=mesh,
        out_type=jax.ShapeDtypeStruct((N_EDGES, EDGE_DIM), jnp.float32),
        scratch_types=[
            pltpu.VMEM((CHUNK,), jnp.int32),
            pltpu.VMEM((CHUNK, EDGE_DIM), jnp.float32),
            pltpu.SemaphoreType.DMA,
        ],
        compiler_params=pltpu.CompilerParams(use_tc_tiling_on_sc=False),
    )
    def k(table_hbm, idx_hbm, out_hbm, idx_v, rows_v, sem):
        wid = lax.axis_index("s") * 2 + lax.axis_index("c")
        base = wid * BASE_CHUNKS + jnp.minimum(wid, EXTRA)
        n = BASE_CHUNKS + (wid < EXTRA).astype(jnp.int32)

        def body(i, carry):
            c = base + i
            pltpu.sync_copy(idx_hbm.at[c], idx_v)
            pltpu.async_copy(table_hbm.at[idx_v], rows_v, sem).wait()
            pltpu.sync_copy(rows_v, out_hbm.at[pl.ds(c * CHUNK, CHUNK)])
            return carry

        lax.fori_loop(0, n, body, 0)

    return k(edge_table, idx2d)


def kernel(x_features, atoms, edge_attr, pos, atom_table, edge_table, W, b):
    del pos
    x_out = _node_proj(x_features, atoms.astype(jnp.int32), atom_table, W, b)
    edge_embed = _edge_gather(edge_table, edge_attr.astype(jnp.int32))
    return (x_out, edge_embed)


# speedup vs baseline: 2.2097x; 1.0032x over previous
import functools

import jax
import jax.numpy as jnp
from jax import lax
from jax.experimental import pallas as pl
from jax.experimental.pallas import tpu as pltpu
from jax.experimental.pallas import tpu_sc as plsc

NODE_DIM = 128
FT_DIM = 128
EDGE_DIM = 16
N_NODES = 100000
N_EDGES = 1600000
N_ATOM = 100

BN = 2000
NB = N_NODES // BN

CHUNK = 128
N_CHUNKS = N_EDGES // CHUNK
NW = 32
W_CHUNKS = N_CHUNKS // NW
EXTRA = N_CHUNKS - W_CHUNKS * NW
K = 13
GROUPS = W_CHUNKS // K
GROUP_ROWS = K * CHUNK
GROUP_FLAT = GROUP_ROWS * EDGE_DIM
CHUNK_FLAT = CHUNK * EDGE_DIM


def _node_body(x_ref, atoms_ref, at_ref, w_ref, b_ref, o_ref):
    a = atoms_ref[0, 0, :]
    onehot = (a[:, None] == lax.broadcasted_iota(jnp.int32, (BN, N_ATOM), 1)
              ).astype(jnp.float32)
    w = w_ref[...]
    p = lax.dot_general(at_ref[...], w[:, :NODE_DIM],
                        (((1,), (1,)), ((), ())),
                        preferred_element_type=jnp.float32)
    t1 = lax.dot_general(onehot, p, (((1,), (0,)), ((), ())),
                         preferred_element_type=jnp.float32)
    t2 = lax.dot_general(x_ref[...], w[:, NODE_DIM:],
                         (((1,), (1,)), ((), ())),
                         preferred_element_type=jnp.float32)
    o_ref[...] = t1 + t2 + b_ref[...]


def _node_proj(x_features, atoms, atom_table, W, b):
    atoms3 = atoms.reshape(NB, 1, BN)
    b2 = b.reshape(1, NODE_DIM)
    return pl.pallas_call(
        _node_body,
        grid=(NB,),
        in_specs=[
            pl.BlockSpec((BN, FT_DIM), lambda i: (i, 0)),
            pl.BlockSpec((1, 1, BN), lambda i: (i, 0, 0)),
            pl.BlockSpec((N_ATOM, NODE_DIM), lambda i: (0, 0)),
            pl.BlockSpec((NODE_DIM, NODE_DIM + FT_DIM), lambda i: (0, 0)),
            pl.BlockSpec((1, NODE_DIM), lambda i: (0, 0)),
        ],
        out_specs=pl.BlockSpec((BN, NODE_DIM), lambda i: (i, 0)),
        out_shape=jax.ShapeDtypeStruct((N_NODES, NODE_DIM), jnp.float32),
    )(x_features, atoms3, atom_table, W, b2)


def _edge_gather(edge_table, edge_attr):
    idx2d = edge_attr.reshape(N_CHUNKS, CHUNK)
    mesh = plsc.VectorSubcoreMesh(core_axis_name="c", subcore_axis_name="s")

    @functools.partial(
        pl.kernel,
        mesh=mesh,
        out_type=jax.ShapeDtypeStruct((N_EDGES, EDGE_DIM), jnp.float32),
        scratch_types=[
            pltpu.VMEM((W_CHUNKS + 1, CHUNK), jnp.int32),
            pltpu.VMEM((GROUP_ROWS, EDGE_DIM), jnp.float32),
            pltpu.SemaphoreType.DMA,
        ],
        compiler_params=pltpu.CompilerParams(use_tc_tiling_on_sc=False),
    )
    def k(table_hbm, idx_hbm, out_hbm, idx_v, rows2d, sem):
        wid = lax.axis_index("s") * 2 + lax.axis_index("c")
        cbase = wid * W_CHUNKS + jnp.minimum(wid, EXTRA)

        pltpu.sync_copy(idx_hbm.at[pl.ds(cbase, W_CHUNKS)],
                        idx_v.at[pl.ds(0, W_CHUNKS)])

        def body(g, carry):
            cps = [pltpu.async_copy(table_hbm.at[idx_v.at[g * K + j]],
                                    rows2d.at[pl.ds(j * CHUNK, CHUNK)], sem)
                   for j in range(K)]
            for cp in cps:
                cp.wait()
            pltpu.sync_copy(rows2d,
                            out_hbm.at[pl.ds((cbase + g * K) * CHUNK,
                                             GROUP_ROWS)])
            return carry

        lax.fori_loop(0, GROUPS, body, 0)

        @pl.when(wid < EXTRA)
        def _tail():
            pltpu.sync_copy(idx_hbm.at[cbase + W_CHUNKS], idx_v.at[W_CHUNKS])
            pltpu.async_copy(table_hbm.at[idx_v.at[W_CHUNKS]],
                             rows2d.at[pl.ds(0, CHUNK)], sem).wait()
            pltpu.sync_copy(rows2d.at[pl.ds(0, CHUNK)],
                            out_hbm.at[pl.ds((cbase + W_CHUNKS) * CHUNK,
                                             CHUNK)])

    return k(edge_table, idx2d)


def kernel(x_features, atoms, edge_attr, pos, atom_table, edge_table, W, b):
    del pos
    x_out = _node_proj(x_features, atoms.astype(jnp.int32), atom_table, W, b)
    edge_embed = _edge_gather(edge_table, edge_attr.astype(jnp.int32))
    return (x_out, edge_embed)


# speedup vs baseline: 3.6359x; 1.6454x over previous
import functools

import jax
import jax.numpy as jnp
from jax import lax
from jax.experimental import pallas as pl
from jax.experimental.pallas import tpu as pltpu
from jax.experimental.pallas import tpu_sc as plsc

NODE_DIM = 128
FT_DIM = 128
EDGE_DIM = 16
N_NODES = 100000
N_EDGES = 1600000
N_ATOM = 100

BN = 2000
NB = N_NODES // BN

CHUNK = 128
N_CHUNKS = N_EDGES // CHUNK
NW = 32
W_CHUNKS = N_CHUNKS // NW
EXTRA = N_CHUNKS - W_CHUNKS * NW
K = 13
GROUPS = W_CHUNKS // K
GROUP_ROWS = K * CHUNK
GROUP_FLAT = GROUP_ROWS * EDGE_DIM
CHUNK_FLAT = CHUNK * EDGE_DIM


def _node_body(x_ref, atoms_ref, at_ref, w_ref, b_ref, o_ref):
    a = atoms_ref[0, 0, :]
    onehot = (a[:, None] == lax.broadcasted_iota(jnp.int32, (BN, N_ATOM), 1)
              ).astype(jnp.float32)
    w = w_ref[...]
    p = lax.dot_general(at_ref[...], w[:, :NODE_DIM],
                        (((1,), (1,)), ((), ())),
                        preferred_element_type=jnp.float32)
    t1 = lax.dot_general(onehot, p, (((1,), (0,)), ((), ())),
                         preferred_element_type=jnp.float32)
    t2 = lax.dot_general(x_ref[...], w[:, NODE_DIM:],
                         (((1,), (1,)), ((), ())),
                         preferred_element_type=jnp.float32)
    o_ref[...] = t1 + t2 + b_ref[...]


def _node_proj(x_features, atoms, atom_table, W, b):
    atoms3 = atoms.reshape(NB, 1, BN)
    b2 = b.reshape(1, NODE_DIM)
    return pl.pallas_call(
        _node_body,
        grid=(NB,),
        in_specs=[
            pl.BlockSpec((BN, FT_DIM), lambda i: (i, 0)),
            pl.BlockSpec((1, 1, BN), lambda i: (i, 0, 0)),
            pl.BlockSpec((N_ATOM, NODE_DIM), lambda i: (0, 0)),
            pl.BlockSpec((NODE_DIM, NODE_DIM + FT_DIM), lambda i: (0, 0)),
            pl.BlockSpec((1, NODE_DIM), lambda i: (0, 0)),
        ],
        out_specs=pl.BlockSpec((BN, NODE_DIM), lambda i: (i, 0)),
        out_shape=jax.ShapeDtypeStruct((N_NODES, NODE_DIM), jnp.float32),
    )(x_features, atoms3, atom_table, W, b2)


def _edge_gather(edge_table, edge_attr):
    idx2d = edge_attr.reshape(N_CHUNKS, CHUNK)
    mesh = plsc.VectorSubcoreMesh(core_axis_name="c", subcore_axis_name="s")

    @functools.partial(
        pl.kernel,
        mesh=mesh,
        out_type=jax.ShapeDtypeStruct((N_EDGES, EDGE_DIM), jnp.float32),
        scratch_types=[
            pltpu.VMEM((50, EDGE_DIM), jnp.float32),
            pltpu.VMEM((W_CHUNKS + 1, CHUNK), jnp.int32),
            pltpu.VMEM((GROUP_ROWS, EDGE_DIM), jnp.float32),
            pltpu.SemaphoreType.DMA,
        ],
        compiler_params=pltpu.CompilerParams(use_tc_tiling_on_sc=False,
                                             needs_layout_passes=False),
    )
    def k(table_hbm, idx_hbm, out_hbm, tbl_v, idx_v, rows2d, sem):
        wid = lax.axis_index("s") * 2 + lax.axis_index("c")
        cbase = wid * W_CHUNKS + jnp.minimum(wid, EXTRA)
        lanes = lax.iota(jnp.int32, 16)
        cols = [jnp.full((16,), j, jnp.int32) for j in range(EDGE_DIM)]

        pltpu.sync_copy(table_hbm, tbl_v)
        pltpu.sync_copy(idx_hbm.at[pl.ds(cbase, W_CHUNKS)],
                        idx_v.at[pl.ds(0, W_CHUNKS)])

        def gather16(chunk, i, rowbase):
            e = idx_v[chunk, pl.ds(i * 16, 16)]
            erow = rowbase + lanes
            for j in range(EDGE_DIM):
                v = plsc.load_gather(tbl_v, [e, cols[j]])
                plsc.store_scatter(rows2d, [erow, cols[j]], v)

        def chunk_body(c2, g):
            def sub(i, _):
                gather16(g * K + c2, i, c2 * CHUNK + i * 16)
                return 0
            lax.fori_loop(0, CHUNK // 16, sub, 0)
            return g

        def body(g, carry):
            lax.fori_loop(0, K, chunk_body, g)
            pltpu.sync_copy(rows2d,
                            out_hbm.at[pl.ds((cbase + g * K) * CHUNK,
                                             GROUP_ROWS)])
            return carry

        lax.fori_loop(0, GROUPS, body, 0)

        @pl.when(wid < EXTRA)
        def _tail():
            pltpu.sync_copy(idx_hbm.at[cbase + W_CHUNKS], idx_v.at[W_CHUNKS])

            def sub(i, _):
                gather16(W_CHUNKS, i, i * 16)
                return 0
            lax.fori_loop(0, CHUNK // 16, sub, 0)
            pltpu.sync_copy(rows2d.at[pl.ds(0, CHUNK)],
                            out_hbm.at[pl.ds((cbase + W_CHUNKS) * CHUNK,
                                             CHUNK)])

    return k(edge_table, idx2d)


def kernel(x_features, atoms, edge_attr, pos, atom_table, edge_table, W, b):
    del pos
    x_out = _node_proj(x_features, atoms.astype(jnp.int32), atom_table, W, b)
    edge_embed = _edge_gather(edge_table, edge_attr.astype(jnp.int32))
    return (x_out, edge_embed)
